# baseline (speedup 1.0000x reference)
import jax
import jax.numpy as jnp
from jax.experimental import pallas as pl
from jax.experimental.pallas import tpu as pltpu

H = W = 50
HP = H + 2
WS = 56
CIN = 512
C3 = 3 * CIN
NROW = HP * WS
NOUT = H * WS
CREG = 36
CCLS = 18
CHEAD = CREG + CCLS


def _rpn_body(xcat_ref, w3_ref, bsw_ref, whead_ref, bhead_ref, out_ref):
    x = xcat_ref[...]
    acc = jnp.dot(x[0:NOUT], w3_ref[0],
                  preferred_element_type=jnp.float32)
    acc += jnp.dot(x[WS:WS + NOUT], w3_ref[1],
                   preferred_element_type=jnp.float32)
    acc += jnp.dot(x[2 * WS:2 * WS + NOUT], w3_ref[2],
                   preferred_element_type=jnp.float32)
    feat = jnp.maximum(acc + bsw_ref[...], 0.0).astype(jnp.bfloat16)
    out = jnp.dot(feat, whead_ref[...],
                  preferred_element_type=jnp.float32) + bhead_ref[...]
    out_ref[...] = out


def kernel(x, W_sw, b_sw, W_cls, b_cls, W_reg, b_reg):
    xh = x[0].transpose(1, 2, 0)
    xp = jnp.pad(xh, ((1, 1), (1, WS + 1 - W), (0, 0)))
    xcat = jnp.concatenate(
        [xp[:, 0:WS], xp[:, 1:WS + 1], xp[:, 2:WS + 2]], axis=-1)
    xcat = xcat.reshape(NROW, C3).astype(jnp.bfloat16)

    w3 = jnp.transpose(W_sw, (2, 3, 1, 0)).reshape(3, C3, CIN)
    w3 = w3.astype(jnp.bfloat16)
    whead = jnp.concatenate(
        [W_reg[:, :, 0, 0], W_cls[:, :, 0, 0]], axis=0).T
    whead = whead.astype(jnp.bfloat16)
    bsw = b_sw.reshape(1, CIN)
    bhead = jnp.concatenate([b_reg, b_cls]).reshape(1, CHEAD)

    out = pl.pallas_call(
        _rpn_body,
        out_shape=jax.ShapeDtypeStruct((NOUT, CHEAD), jnp.float32),
        in_specs=[
            pl.BlockSpec(memory_space=pltpu.VMEM),
            pl.BlockSpec(memory_space=pltpu.VMEM),
            pl.BlockSpec(memory_space=pltpu.VMEM),
            pl.BlockSpec(memory_space=pltpu.VMEM),
            pl.BlockSpec(memory_space=pltpu.VMEM),
        ],
        out_specs=pl.BlockSpec(memory_space=pltpu.VMEM),
    )(xcat, w3, bsw, whead, bhead)

    out = out.reshape(H, WS, CHEAD)[:, :W, :]
    reg = out[:, :, :CREG].reshape(1, H * W * 9, 4)
    cls = out[:, :, CREG:].reshape(1, H * W * 9, 2)
    return (reg, cls)
